# baseline (speedup 1.0000x reference)
import functools

import jax
import jax.numpy as jnp
from jax import lax
from jax.experimental import pallas as pl
from jax.experimental.pallas import tpu as pltpu
from jax.experimental.pallas import tpu_sc as plsc

NUM_NODES = 1000000
D = 64
B = 16384
K = 5
NC = 2
NS = 16
NW = NC * NS
BT = B // NW
CB = 128
NCHUNK = BT // CB
L = 16


def _sc_scores_body(vert_hbm, ctx_hbm, tgt_hbm, ctxn_hbm, negt_hbm,
                    pos_out, neg_out,
                    tidx_v, cidx_v, nidx_v, vrows_v, crows_v, nrows_v,
                    psc_v, nsc_v, sem):
    wid = lax.axis_index("s") * NC + lax.axis_index("c")
    base0 = wid * BT
    lane = lax.iota(jnp.int32, L)

    for chunk in range(NCHUNK):
        base = base0 + chunk * CB
        pltpu.sync_copy(tgt_hbm.at[pl.ds(base, CB)], tidx_v)
        pltpu.sync_copy(ctxn_hbm.at[pl.ds(base, CB)], cidx_v)
        for k in range(K):
            pltpu.sync_copy(negt_hbm.at[pl.ds(k * B + base, CB)], nidx_v[k])
        cps = [pltpu.async_copy(vert_hbm.at[tidx_v], vrows_v, sem),
               pltpu.async_copy(ctx_hbm.at[cidx_v], crows_v, sem)]
        for k in range(K):
            cps.append(pltpu.async_copy(ctx_hbm.at[nidx_v[k]], nrows_v[k], sem))
        for cp in cps:
            cp.wait()

        for g in range(CB // L):
            row = g * L + lane

            def jbody(j, accs, row=row):
                col = jnp.full((L,), j, jnp.int32)
                vg = plsc.load_gather(vrows_v, [row, col])
                cg = plsc.load_gather(crows_v, [row, col])
                out = [accs[0] + vg * cg]
                for k in range(K):
                    ng = plsc.load_gather(nrows_v[k], [row, col])
                    out.append(accs[1 + k] + vg * ng)
                return tuple(out)

            zero = jnp.zeros((L,), jnp.float32)
            accs = lax.fori_loop(0, D, jbody, (zero,) * (K + 1))
            psc_v[pl.ds(g * L, L)] = accs[0]
            for k in range(K):
                nsc_v[k][pl.ds(g * L, L)] = accs[1 + k]

        pltpu.sync_copy(psc_v, pos_out.at[pl.ds(base, CB)])
        for k in range(K):
            pltpu.sync_copy(nsc_v[k], neg_out.at[pl.ds(k * B + base, CB)])


def _sc_scores(vertex_emb, context_emb, target_nodes, context_nodes, neg_t):
    mesh = plsc.VectorSubcoreMesh(core_axis_name="c", subcore_axis_name="s",
                                  num_cores=NC, num_subcores=NS)
    return pl.kernel(
        _sc_scores_body,
        out_type=(jax.ShapeDtypeStruct((B,), jnp.float32),
                  jax.ShapeDtypeStruct((K * B,), jnp.float32)),
        mesh=mesh,
        compiler_params=pltpu.CompilerParams(use_tc_tiling_on_sc=False,
                                             needs_layout_passes=False),
        scratch_types=(
            pltpu.VMEM((CB,), jnp.int32),
            pltpu.VMEM((CB,), jnp.int32),
            [pltpu.VMEM((CB,), jnp.int32) for _ in range(K)],
            pltpu.VMEM((CB, D), jnp.float32),
            pltpu.VMEM((CB, D), jnp.float32),
            [pltpu.VMEM((CB, D), jnp.float32) for _ in range(K)],
            pltpu.VMEM((CB,), jnp.float32),
            [pltpu.VMEM((CB,), jnp.float32) for _ in range(K)],
            pltpu.SemaphoreType.DMA,
        ),
    )(vertex_emb, context_emb, target_nodes, context_nodes, neg_t)


def _loss_body(pos_ref, neg_ref, out_ref):
    def logsig(x):
        return jnp.minimum(x, 0.0) - jnp.log(1.0 + jnp.exp(-jnp.abs(x)))

    total = jnp.sum(logsig(pos_ref[...])) + jnp.sum(logsig(-neg_ref[...]))
    out_ref[0, 0] = -total / B


def _tc_loss(pos_s, neg_s):
    out = pl.pallas_call(
        _loss_body,
        out_shape=jax.ShapeDtypeStruct((1, 1), jnp.float32),
        in_specs=[pl.BlockSpec(memory_space=pltpu.VMEM),
                  pl.BlockSpec(memory_space=pltpu.VMEM)],
        out_specs=pl.BlockSpec(memory_space=pltpu.SMEM),
    )(pos_s.reshape(B // 128, 128), neg_s.reshape(K * B // 128, 128))
    return out[0, 0]


@jax.jit
def kernel(vertex_emb, context_emb, target_nodes, context_nodes, neg_nodes):
    neg_t = neg_nodes.astype(jnp.int32).T.reshape(K * B)
    pos_s, neg_s = _sc_scores(vertex_emb, context_emb,
                              target_nodes.astype(jnp.int32),
                              context_nodes.astype(jnp.int32), neg_t)
    return _tc_loss(pos_s, neg_s)
